# baseline (speedup 1.0000x reference)
import functools
import jax, jax.numpy as jnp
from jax import lax
from jax.experimental import pallas as pl
from jax.experimental.pallas import tpu as pltpu

_VOCAB = 100000
_EMB = 64
_CTX = 10
_HID = 128
_MAIN = 98304
_TAIL = _VOCAB - _MAIN
_TPAD = 2048
_VPAD = _MAIN + _TPAD
_BC = 12800
_CHUNKS = [_BC] * 7 + [_MAIN - 7 * _BC]
_NORM_BC = 4096


def _body(idx_ref, emb_ref, W1_ref, b1_ref, W2_ref, b2_ref, w2t_ref, b2t_ref,
          out_ref, xg_ref, bufs_ref, sems_ref, gsem_ref):
    offs = [sum(_CHUNKS[:c]) for c in range(len(_CHUNKS))]

    def w2_copy(c, b):
        return pltpu.make_async_copy(
            W2_ref.at[:, pl.ds(offs[c], _CHUNKS[c])],
            bufs_ref.at[b, :, pl.ds(0, _CHUNKS[c])],
            sems_ref.at[b],
        )

    w2_copy(0, 0).start()
    w2_copy(1, 1).start()
    gathers = [
        pltpu.make_async_copy(
            emb_ref.at[pl.ds(idx_ref[r], 1), :],
            xg_ref.at[pl.ds(r, 1), :],
            gsem_ref,
        )
        for r in range(2 * _CTX)
    ]
    for g in gathers:
        g.start()
    for g in gathers:
        g.wait()

    h = b1_ref[...]
    for r in range(2 * _CTX):
        h = h + jnp.dot(xg_ref[pl.ds(r, 1), :], W1_ref[r],
                        preferred_element_type=jnp.float32)
    h16 = jnp.maximum(h, 0.0).astype(jnp.bfloat16)

    m = jnp.float32(-3.0e38)
    s = jnp.float32(0.0)
    for c in range(len(_CHUNKS)):
        b = c % 2
        w2_copy(c, b).wait()
        z = jnp.dot(h16, bufs_ref[b, :, pl.ds(0, _CHUNKS[c])].astype(jnp.bfloat16),
                    preferred_element_type=jnp.float32)
        if c + 2 < len(_CHUNKS):
            w2_copy(c + 2, b).start()
        z = z + b2_ref[:, pl.ds(offs[c], _CHUNKS[c])]
        out_ref[:, pl.ds(offs[c], _CHUNKS[c])] = z
        m_new = jnp.maximum(m, jnp.max(z))
        s = s * jnp.exp(m - m_new) + jnp.sum(jnp.exp(z - m_new))
        m = m_new

    zt = jnp.dot(h16, w2t_ref[...].astype(jnp.bfloat16),
                 preferred_element_type=jnp.float32) + b2t_ref[...]
    m_new = jnp.maximum(m, jnp.max(zt))
    s = s * jnp.exp(m - m_new) + jnp.sum(jnp.exp(zt - m_new))
    lse = m_new + jnp.log(s)
    out_ref[:, pl.ds(_MAIN, _TPAD)] = zt - lse

    for c in range(_MAIN // _NORM_BC):
        sl = pl.ds(c * _NORM_BC, _NORM_BC)
        out_ref[:, sl] = out_ref[:, sl] - lse


def kernel(inputs, emb, W1, b1, W2, b2):
    idx = jnp.asarray(inputs, jnp.int32)
    W1r = W1.reshape(2 * _CTX, _EMB, _HID)
    b1r = b1.reshape(1, _HID)
    b2r = b2.reshape(1, _VOCAB)
    w2t = jnp.pad(lax.slice(W2, (0, _MAIN), (_HID, _VOCAB)),
                  ((0, 0), (0, _TPAD - _TAIL)))
    b2t = jnp.pad(lax.slice(b2r, (0, _MAIN), (1, _VOCAB)),
                  ((0, 0), (0, _TPAD - _TAIL)), constant_values=-3.0e38)

    grid_spec = pltpu.PrefetchScalarGridSpec(
        num_scalar_prefetch=1,
        grid=(1,),
        in_specs=[
            pl.BlockSpec(memory_space=pltpu.HBM),
            pl.BlockSpec((2 * _CTX, _EMB, _HID), lambda i, idx_ref: (0, 0, 0)),
            pl.BlockSpec((1, _HID), lambda i, idx_ref: (0, 0)),
            pl.BlockSpec(memory_space=pltpu.HBM),
            pl.BlockSpec((1, _VOCAB), lambda i, idx_ref: (0, 0)),
            pl.BlockSpec((_HID, _TPAD), lambda i, idx_ref: (0, 0)),
            pl.BlockSpec((1, _TPAD), lambda i, idx_ref: (0, 0)),
        ],
        out_specs=pl.BlockSpec((1, _VPAD), lambda i, idx_ref: (0, 0)),
        scratch_shapes=[
            pltpu.VMEM((2 * _CTX, _EMB), jnp.float32),
            pltpu.VMEM((2, _HID, _BC), jnp.float32),
            pltpu.SemaphoreType.DMA((2,)),
            pltpu.SemaphoreType.DMA,
        ],
    )

    out = pl.pallas_call(
        _body,
        grid_spec=grid_spec,
        out_shape=jax.ShapeDtypeStruct((1, _VPAD), jnp.float32),
        compiler_params=pltpu.CompilerParams(
            vmem_limit_bytes=100 * 1024 * 1024,
        ),
    )(idx, emb, W1r, b1r, W2, b2r, w2t, b2t)
    return out[:, :_VOCAB]
